# baseline (speedup 1.0000x reference)
import jax
import jax.numpy as jnp
from jax.experimental import pallas as pl
from jax.experimental.pallas import tpu as pltpu

_GRID = 2


def _gather_block_kernel(idx_ref, lut_ref, out_ref, sem):
    blk = out_ref.shape[0]
    base = pl.program_id(0) * blk
    for b in range(blk):
        pltpu.make_async_copy(
            lut_ref.at[pl.ds(idx_ref[base + b], 1), :],
            out_ref.at[pl.ds(b, 1), :],
            sem,
        ).start()
    pltpu.make_async_copy(
        lut_ref.at[pl.ds(0, blk), :],
        out_ref.at[pl.ds(0, blk), :],
        sem,
    ).wait()


def kernel(task_indices, fused_table):
    B = task_indices.shape[0]
    D = fused_table.shape[1]
    blk = B // _GRID
    return pl.pallas_call(
        _gather_block_kernel,
        out_shape=jax.ShapeDtypeStruct((B, D), fused_table.dtype),
        grid=(_GRID,),
        in_specs=[
            pl.BlockSpec(memory_space=pltpu.MemorySpace.SMEM),
            pl.BlockSpec(memory_space=pl.ANY),
        ],
        out_specs=pl.BlockSpec((blk, D), lambda i: (i, 0)),
        scratch_shapes=[pltpu.SemaphoreType.DMA],
        compiler_params=pltpu.CompilerParams(
            dimension_semantics=("parallel",),
        ),
    )(task_indices.astype(jnp.int32), fused_table)


# speedup vs baseline: 1.3867x; 1.3867x over previous
import jax
import jax.numpy as jnp
from jax.experimental import pallas as pl
from jax.experimental.pallas import tpu as pltpu


def _gather_kernel(idx_ref, lut_ref, out_ref, buf, sem_a, sem_b, out_sem):
    B = out_ref.shape[0]
    H = B // 2
    for b in range(B):
        pltpu.make_async_copy(
            lut_ref.at[pl.ds(idx_ref[b], 1), :],
            buf.at[pl.ds(b, 1), :],
            sem_a if b < H else sem_b,
        ).start()
    pltpu.make_async_copy(
        lut_ref.at[pl.ds(0, H), :], buf.at[pl.ds(0, H), :], sem_a
    ).wait()
    pltpu.make_async_copy(
        buf.at[pl.ds(0, H), :], out_ref.at[pl.ds(0, H), :], out_sem
    ).start()
    pltpu.make_async_copy(
        lut_ref.at[pl.ds(0, H), :], buf.at[pl.ds(0, H), :], sem_b
    ).wait()
    pltpu.make_async_copy(
        buf.at[pl.ds(H, H), :], out_ref.at[pl.ds(H, H), :], out_sem
    ).start()
    pltpu.make_async_copy(
        buf.at[pl.ds(0, B), :], out_ref.at[pl.ds(0, B), :], out_sem
    ).wait()


def kernel(task_indices, fused_table):
    B = task_indices.shape[0]
    D = fused_table.shape[1]
    return pl.pallas_call(
        _gather_kernel,
        out_shape=jax.ShapeDtypeStruct((B, D), fused_table.dtype),
        in_specs=[
            pl.BlockSpec(memory_space=pltpu.MemorySpace.SMEM),
            pl.BlockSpec(memory_space=pl.ANY),
        ],
        out_specs=pl.BlockSpec(memory_space=pl.ANY),
        scratch_shapes=[
            pltpu.VMEM((B, D), jnp.float32),
            pltpu.SemaphoreType.DMA,
            pltpu.SemaphoreType.DMA,
            pltpu.SemaphoreType.DMA,
        ],
    )(task_indices.astype(jnp.int32), fused_table)
